# Initial kernel scaffold; baseline (speedup 1.0000x reference)
import jax
import jax.numpy as jnp
from jax.experimental import pallas as pl


def kernel(right_embed, edge_index, adj_vals, perm, gcnW1, highwayWr, highwaybr):
    raise NotImplementedError("write your pallas kernel here")


# speedup vs baseline: 4.0246x; 4.0246x over previous
import functools

import jax
import jax.numpy as jnp
from jax import lax
from jax.experimental import pallas as pl
from jax.experimental.pallas import tpu as pltpu
from jax.experimental.pallas import tpu_sc as plsc

N = 10000
E = 320000
D = 128

NC = 2
NS = 16
NW = NC * NS

EPT = E // NW
CH = 80
NCHUNK = EPT // CH

RPS = 624
RREM = N - NS * RPS
ROWS_PT = 312
LG_REM = N - NW * ROWS_PT


def _tc_matmul(x, w):
    def body(x_ref, w_ref, o_ref):
        o_ref[...] = jnp.dot(x_ref[...], w_ref[...],
                             preferred_element_type=jnp.float32)

    return pl.pallas_call(
        body,
        grid=(N // 1000,),
        in_specs=[pl.BlockSpec((1000, D), lambda i: (i, 0)),
                  pl.BlockSpec((D, D), lambda i: (0, 0))],
        out_specs=pl.BlockSpec((1000, D), lambda i: (i, 0)),
        out_shape=jax.ShapeDtypeStruct((N, D), jnp.float32),
    )(x, w)


def _sc_spmm(h, rows3, cols3, vals3, perm, re, zblk):
    mesh = plsc.VectorSubcoreMesh(core_axis_name="c", subcore_axis_name="s")

    @functools.partial(
        pl.kernel,
        mesh=mesh,
        out_type=[
            jax.ShapeDtypeStruct((N, D), jnp.float32),
            jax.ShapeDtypeStruct((N, D), jnp.float32),
            jax.ShapeDtypeStruct((N, D), jnp.float32),
        ],
        scratch_types=[
            pltpu.VMEM((CH,), jnp.int32),
            pltpu.VMEM((CH,), jnp.int32),
            pltpu.VMEM((CH,), jnp.float32),
            pltpu.VMEM((CH, D), jnp.float32),
            pltpu.VMEM_SHARED((N, D), jnp.float32),
        ],
    )
    def body(h_hbm, rows_hbm, cols_hbm, vals_hbm, perm_hbm, re_hbm, z_hbm,
             part0_hbm, part1_hbm, left_hbm,
             ridx_v, cidx_v, vals_v, gat_v, acc_sh):
        c = lax.axis_index("c")
        s = lax.axis_index("s")
        wid = s * NC + c

        pltpu.sync_copy(z_hbm, acc_sh.at[pl.ds(s * RPS, RPS)])

        @pl.when(s == NS - 1)
        def _():
            pltpu.sync_copy(z_hbm.at[pl.ds(0, RREM)],
                            acc_sh.at[pl.ds(NS * RPS, RREM)])

        for t, lg in enumerate((80, 80, 80, 72)):
            base = wid * ROWS_PT + t * 80
            pltpu.sync_copy(perm_hbm.at[pl.ds(base, lg)],
                            cidx_v.at[pl.ds(0, lg)])
            pltpu.sync_copy(re_hbm.at[cidx_v.at[pl.ds(0, lg)]],
                            gat_v.at[pl.ds(0, lg)])
            pltpu.sync_copy(gat_v.at[pl.ds(0, lg)],
                            left_hbm.at[pl.ds(base, lg)])

        @pl.when(wid == 0)
        def _():
            pltpu.sync_copy(perm_hbm.at[pl.ds(NW * ROWS_PT, LG_REM)],
                            cidx_v.at[pl.ds(0, LG_REM)])
            pltpu.sync_copy(re_hbm.at[cidx_v.at[pl.ds(0, LG_REM)]],
                            gat_v.at[pl.ds(0, LG_REM)])
            pltpu.sync_copy(gat_v.at[pl.ds(0, LG_REM)],
                            left_hbm.at[pl.ds(NW * ROWS_PT, LG_REM)])

        plsc.subcore_barrier()

        def chunk_body(j, carry):
            eb = wid * NCHUNK + j
            pltpu.sync_copy(rows_hbm.at[eb], ridx_v)
            pltpu.sync_copy(cols_hbm.at[eb], cidx_v)
            pltpu.sync_copy(vals_hbm.at[eb], vals_v)
            pltpu.sync_copy(h_hbm.at[cidx_v], gat_v)

            def group_body(g, carry2):
                vgroup = vals_v[pl.ds(g * 16, 16)]
                for k in range(16):
                    v16 = jnp.broadcast_to(vgroup[k], (16,))
                    e = g * 16 + k
                    for q in range(D // 16):
                        sl = pl.ds(q * 16, 16)
                        gat_v[e, sl] = gat_v[e, sl] * v16
                return carry2

            lax.fori_loop(0, CH // 16, group_body, 0)
            pltpu.sync_copy(gat_v, acc_sh.at[ridx_v], add=True)
            return carry

        lax.fori_loop(0, NCHUNK, chunk_body, 0)

        plsc.subcore_barrier()

        @pl.when(c == 0)
        def _():
            pltpu.sync_copy(acc_sh.at[pl.ds(s * RPS, RPS)],
                            part0_hbm.at[pl.ds(s * RPS, RPS)])

        @pl.when(c == 1)
        def _():
            pltpu.sync_copy(acc_sh.at[pl.ds(s * RPS, RPS)],
                            part1_hbm.at[pl.ds(s * RPS, RPS)])

        @pl.when((s == NS - 1) & (c == 0))
        def _():
            pltpu.sync_copy(acc_sh.at[pl.ds(NS * RPS, RREM)],
                            part0_hbm.at[pl.ds(NS * RPS, RREM)])

        @pl.when((s == NS - 1) & (c == 1))
        def _():
            pltpu.sync_copy(acc_sh.at[pl.ds(NS * RPS, RREM)],
                            part1_hbm.at[pl.ds(NS * RPS, RREM)])

    return body(h, rows3, cols3, vals3, perm, re, zblk)


def _tc_final(p0, p1, left, w, b):
    def body(p0_ref, p1_ref, l_ref, w_ref, b_ref, o_ref):
        lft = l_ref[...]
        g = jax.nn.sigmoid(
            jnp.dot(lft, w_ref[...], preferred_element_type=jnp.float32)
            + b_ref[...])
        p = jnp.maximum(p0_ref[...] + p1_ref[...], 0.0)
        o_ref[...] = g * p + (1.0 - g) * lft

    return pl.pallas_call(
        body,
        grid=(N // 1000,),
        in_specs=[pl.BlockSpec((1000, D), lambda i: (i, 0)),
                  pl.BlockSpec((1000, D), lambda i: (i, 0)),
                  pl.BlockSpec((1000, D), lambda i: (i, 0)),
                  pl.BlockSpec((D, D), lambda i: (0, 0)),
                  pl.BlockSpec((1, D), lambda i: (0, 0))],
        out_specs=pl.BlockSpec((1000, D), lambda i: (i, 0)),
        out_shape=jax.ShapeDtypeStruct((N, D), jnp.float32),
    )(p0, p1, left, w, b)


def kernel(right_embed, edge_index, adj_vals, perm, gcnW1,
           highwayWr, highwaybr):
    rows3 = edge_index[0].astype(jnp.int32).reshape(NW * NCHUNK, CH)
    cols3 = edge_index[1].astype(jnp.int32).reshape(NW * NCHUNK, CH)
    vals3 = adj_vals.reshape(NW * NCHUNK, CH)
    zblk = jnp.zeros((RPS, D), jnp.float32)

    h = _tc_matmul(right_embed, gcnW1)
    part0, part1, left = _sc_spmm(h, rows3, cols3, vals3,
                                  perm.astype(jnp.int32), right_embed, zblk)
    return _tc_final(part0, part1, left, highwayWr,
                     highwaybr.reshape(1, D))


# speedup vs baseline: 5.7281x; 1.4233x over previous
import functools

import jax
import jax.numpy as jnp
from jax import lax
from jax.experimental import pallas as pl
from jax.experimental.pallas import tpu as pltpu
from jax.experimental.pallas import tpu_sc as plsc

N = 10000
E = 320000
D = 128

NC = 2
NS = 16
NW = NC * NS

EPT = E // NW
CH = 80
NCHUNK = EPT // CH

RPS = 624
RREM = N - NS * RPS
ROWS_PT = 312
LG_REM = N - NW * ROWS_PT


def _tc_matmul(x, w):
    def body(x_ref, w_ref, o_ref):
        o_ref[...] = jnp.dot(x_ref[...], w_ref[...],
                             preferred_element_type=jnp.float32)

    return pl.pallas_call(
        body,
        grid=(N // 1000,),
        in_specs=[pl.BlockSpec((1000, D), lambda i: (i, 0)),
                  pl.BlockSpec((D, D), lambda i: (0, 0))],
        out_specs=pl.BlockSpec((1000, D), lambda i: (i, 0)),
        out_shape=jax.ShapeDtypeStruct((N, D), jnp.float32),
    )(x, w)


def _sc_spmm(h, rc2, vals2, perm, re, zblk):
    mesh = plsc.VectorSubcoreMesh(core_axis_name="c", subcore_axis_name="s")

    @functools.partial(
        pl.kernel,
        mesh=mesh,
        out_type=[
            jax.ShapeDtypeStruct((N, D), jnp.float32),
            jax.ShapeDtypeStruct((N, D), jnp.float32),
            jax.ShapeDtypeStruct((N, D), jnp.float32),
        ],
        scratch_types=[
            pltpu.VMEM((2, CH), jnp.int32),
            pltpu.VMEM((2, CH), jnp.int32),
            pltpu.VMEM((2, CH), jnp.int32),
            pltpu.VMEM((CH,), jnp.float32),
            pltpu.VMEM((CH,), jnp.float32),
            pltpu.VMEM((CH,), jnp.float32),
            pltpu.VMEM((CH, D), jnp.float32),
            pltpu.VMEM((CH, D), jnp.float32),
            pltpu.VMEM((CH, D), jnp.float32),
            pltpu.VMEM_SHARED((N, D), jnp.float32),
            pltpu.SemaphoreType.DMA,
            pltpu.SemaphoreType.DMA,
            pltpu.SemaphoreType.DMA,
            pltpu.SemaphoreType.DMA,
            pltpu.SemaphoreType.DMA,
            pltpu.SemaphoreType.DMA,
        ],
    )
    def body(h_hbm, rc_hbm, vals_hbm, perm_hbm, re_hbm, z_hbm,
             part0_hbm, part1_hbm, left_hbm,
             rc0, rc1, rc2, val0, val1, val2, gat0, gat1, gat2, acc_sh,
             sg0, sg1, sg2, ss0, ss1, ss2):
        c = lax.axis_index("c")
        s = lax.axis_index("s")
        wid = s * NC + c
        rcs = (rc0, rc1, rc2)
        vls = (val0, val1, val2)
        gats = (gat0, gat1, gat2)
        sgs = (sg0, sg1, sg2)
        sss = (ss0, ss1, ss2)

        pltpu.sync_copy(z_hbm, acc_sh.at[pl.ds(s * RPS, RPS)])

        @pl.when(s == NS - 1)
        def _():
            pltpu.sync_copy(z_hbm.at[pl.ds(0, RREM)],
                            acc_sh.at[pl.ds(NS * RPS, RREM)])

        for t, lg in enumerate((80, 80, 80, 72)):
            base = wid * ROWS_PT + t * 80
            pltpu.sync_copy(perm_hbm.at[pl.ds(base, lg)],
                            rc0.at[0, pl.ds(0, lg)])
            pltpu.sync_copy(re_hbm.at[rc0.at[0, pl.ds(0, lg)]],
                            gat0.at[pl.ds(0, lg)])
            pltpu.sync_copy(gat0.at[pl.ds(0, lg)],
                            left_hbm.at[pl.ds(base, lg)])

        @pl.when(wid == 0)
        def _():
            pltpu.sync_copy(perm_hbm.at[pl.ds(NW * ROWS_PT, LG_REM)],
                            rc0.at[0, pl.ds(0, LG_REM)])
            pltpu.sync_copy(re_hbm.at[rc0.at[0, pl.ds(0, LG_REM)]],
                            gat0.at[pl.ds(0, LG_REM)])
            pltpu.sync_copy(gat0.at[pl.ds(0, LG_REM)],
                            left_hbm.at[pl.ds(NW * ROWS_PT, LG_REM)])

        plsc.subcore_barrier()

        ebase = wid * NCHUNK

        def load_edat(eb, r):
            pltpu.sync_copy(rc_hbm.at[eb], rcs[r])
            pltpu.sync_copy(vals_hbm.at[eb], vls[r])

        def start_gather(r):
            pltpu.async_copy(h_hbm.at[rcs[r].at[1]], gats[r], sgs[r])

        def wait_gather(r):
            pltpu.make_async_copy(h_hbm.at[rcs[r].at[1]], gats[r],
                                  sgs[r]).wait()

        def start_scatter(r):
            pltpu.async_copy(gats[r], acc_sh.at[rcs[r].at[0]], sss[r],
                             add=True)

        def wait_scatter(r):
            pltpu.make_async_copy(gats[r], acc_sh.at[rcs[r].at[0]],
                                  sss[r]).wait()

        def scale(r):
            gat = gats[r]
            vv = vls[r]

            def group_body(g, carry2):
                vgroup = vv[pl.ds(g * 16, 16)]
                for k in range(16):
                    v16 = jnp.broadcast_to(vgroup[k], (16,))
                    e = g * 16 + k
                    for q in range(D // 16):
                        sl = pl.ds(q * 16, 16)
                        gat[e, sl] = gat[e, sl] * v16
                return carry2

            lax.fori_loop(0, CH // 16, group_body, 0)

        load_edat(ebase, 0)
        start_gather(0)
        wait_gather(0)
        load_edat(ebase + 1, 1)
        start_gather(1)
        scale(0)
        start_scatter(0)
        wait_gather(1)
        load_edat(ebase + 2, 2)
        start_gather(2)
        scale(1)
        start_scatter(1)

        def pipe_body(i, carry):
            j = 3 * i + 2
            for cr in (2, 0, 1):
                nr = (cr + 1) % 3
                wait_gather(cr)
                wait_scatter(nr)
                eb = ebase + jnp.minimum(j + 1, NCHUNK - 1)
                load_edat(eb, nr)
                start_gather(nr)
                scale(cr)
                start_scatter(cr)
                j = j + 1
            return carry

        lax.fori_loop(0, (NCHUNK - 2) // 3, pipe_body, 0)

        wait_scatter(0)
        wait_scatter(1)
        wait_gather(2)

        plsc.subcore_barrier()

        @pl.when(c == 0)
        def _():
            pltpu.sync_copy(acc_sh.at[pl.ds(s * RPS, RPS)],
                            part0_hbm.at[pl.ds(s * RPS, RPS)])

        @pl.when(c == 1)
        def _():
            pltpu.sync_copy(acc_sh.at[pl.ds(s * RPS, RPS)],
                            part1_hbm.at[pl.ds(s * RPS, RPS)])

        @pl.when((s == NS - 1) & (c == 0))
        def _():
            pltpu.sync_copy(acc_sh.at[pl.ds(NS * RPS, RREM)],
                            part0_hbm.at[pl.ds(NS * RPS, RREM)])

        @pl.when((s == NS - 1) & (c == 1))
        def _():
            pltpu.sync_copy(acc_sh.at[pl.ds(NS * RPS, RREM)],
                            part1_hbm.at[pl.ds(NS * RPS, RREM)])

    return body(h, rc2, vals2, perm, re, zblk)


def _tc_final(p0, p1, left, w, b):
    def body(p0_ref, p1_ref, l_ref, w_ref, b_ref, o_ref):
        lft = l_ref[...]
        g = jax.nn.sigmoid(
            jnp.dot(lft, w_ref[...], preferred_element_type=jnp.float32)
            + b_ref[...])
        p = jnp.maximum(p0_ref[...] + p1_ref[...], 0.0)
        o_ref[...] = g * p + (1.0 - g) * lft

    return pl.pallas_call(
        body,
        grid=(N // 1000,),
        in_specs=[pl.BlockSpec((1000, D), lambda i: (i, 0)),
                  pl.BlockSpec((1000, D), lambda i: (i, 0)),
                  pl.BlockSpec((1000, D), lambda i: (i, 0)),
                  pl.BlockSpec((D, D), lambda i: (0, 0)),
                  pl.BlockSpec((1, D), lambda i: (0, 0))],
        out_specs=pl.BlockSpec((1000, D), lambda i: (i, 0)),
        out_shape=jax.ShapeDtypeStruct((N, D), jnp.float32),
    )(p0, p1, left, w, b)


def kernel(right_embed, edge_index, adj_vals, perm, gcnW1,
           highwayWr, highwaybr):
    ei = edge_index.astype(jnp.int32)
    rc2 = jnp.stack([ei[0].reshape(NW * NCHUNK, CH),
                     ei[1].reshape(NW * NCHUNK, CH)], axis=1)
    vals2 = adj_vals.reshape(NW * NCHUNK, CH)
    zblk = jnp.zeros((RPS, D), jnp.float32)

    h = _tc_matmul(right_embed, gcnW1)
    part0, part1, left = _sc_spmm(h, rc2, vals2,
                                  perm.astype(jnp.int32), right_embed, zblk)
    return _tc_final(part0, part1, left, highwayWr,
                     highwaybr.reshape(1, D))


# speedup vs baseline: 7.2633x; 1.2680x over previous
import functools

import jax
import jax.numpy as jnp
from jax import lax
from jax.experimental import pallas as pl
from jax.experimental.pallas import tpu as pltpu
from jax.experimental.pallas import tpu_sc as plsc

N = 10000
E = 320000
D = 128

NC = 2
NS = 16
NW = NC * NS

EPT = E // NW
CH = 80
NCHUNK = EPT // CH
SB = 5
NSUPER = NCHUNK // SB

RPS = 624
RREM = N - NS * RPS
ROWS_PT = 312
LG_REM = N - NW * ROWS_PT

STEADY_START = 2
STEADY_ITERS = 4
PERIOD = 30


def _tc_matmul(x, w):
    def body(x_ref, w_ref, o_ref):
        o_ref[...] = jnp.dot(x_ref[...], w_ref[...],
                             preferred_element_type=jnp.float32)

    return pl.pallas_call(
        body,
        grid=(N // 1000,),
        in_specs=[pl.BlockSpec((1000, D), lambda i: (i, 0)),
                  pl.BlockSpec((D, D), lambda i: (0, 0))],
        out_specs=pl.BlockSpec((1000, D), lambda i: (i, 0)),
        out_shape=jax.ShapeDtypeStruct((N, D), jnp.float32),
    )(x, w)


def _coords(j):
    return (j // SB) % 2, j % SB, j % 3


def _sc_spmm(h, rc2, vals2, perm, re, zblk):
    mesh = plsc.VectorSubcoreMesh(core_axis_name="c", subcore_axis_name="s")

    @functools.partial(
        pl.kernel,
        mesh=mesh,
        out_type=[
            jax.ShapeDtypeStruct((N, D), jnp.float32),
            jax.ShapeDtypeStruct((N, D), jnp.float32),
            jax.ShapeDtypeStruct((N, D), jnp.float32),
        ],
        scratch_types=[
            pltpu.VMEM((2 * SB, CH), jnp.int32),
            pltpu.VMEM((2 * SB, CH), jnp.int32),
            pltpu.VMEM((SB, CH + 16), jnp.float32),
            pltpu.VMEM((SB, CH + 16), jnp.float32),
            pltpu.VMEM((CH, D), jnp.float32),
            pltpu.VMEM((CH, D), jnp.float32),
            pltpu.VMEM((CH, D), jnp.float32),
            pltpu.VMEM_SHARED((N, D), jnp.float32),
            pltpu.SemaphoreType.DMA,
            pltpu.SemaphoreType.DMA,
            pltpu.SemaphoreType.DMA,
            pltpu.SemaphoreType.DMA,
            pltpu.SemaphoreType.DMA,
            pltpu.SemaphoreType.DMA,
        ],
    )
    def body(h_hbm, rc_hbm, vals_hbm, perm_hbm, re_hbm, z_hbm,
             part0_hbm, part1_hbm, left_hbm,
             rcA, rcB, valA, valB, gat0, gat1, gat2, acc_sh,
             sg0, sg1, sg2, ss0, ss1, ss2):
        c = lax.axis_index("c")
        s = lax.axis_index("s")
        wid = s * NC + c
        rcs = (rcA, rcB)
        vls = (valA, valB)
        gats = (gat0, gat1, gat2)
        sgs = (sg0, sg1, sg2)
        sss = (ss0, ss1, ss2)

        pltpu.sync_copy(z_hbm, acc_sh.at[pl.ds(s * RPS, RPS)])

        @pl.when(s == NS - 1)
        def _():
            pltpu.sync_copy(z_hbm.at[pl.ds(0, RREM)],
                            acc_sh.at[pl.ds(NS * RPS, RREM)])

        for t, lg in enumerate((80, 80, 80, 72)):
            base = wid * ROWS_PT + t * 80
            pltpu.sync_copy(perm_hbm.at[pl.ds(base, lg)],
                            rcA.at[0, pl.ds(0, lg)])
            pltpu.sync_copy(re_hbm.at[rcA.at[0, pl.ds(0, lg)]],
                            gat0.at[pl.ds(0, lg)])
            pltpu.sync_copy(gat0.at[pl.ds(0, lg)],
                            left_hbm.at[pl.ds(base, lg)])

        @pl.when(wid == 0)
        def _():
            pltpu.sync_copy(perm_hbm.at[pl.ds(NW * ROWS_PT, LG_REM)],
                            rcA.at[0, pl.ds(0, LG_REM)])
            pltpu.sync_copy(re_hbm.at[rcA.at[0, pl.ds(0, LG_REM)]],
                            gat0.at[pl.ds(0, LG_REM)])
            pltpu.sync_copy(gat0.at[pl.ds(0, LG_REM)],
                            left_hbm.at[pl.ds(NW * ROWS_PT, LG_REM)])

        plsc.subcore_barrier()

        def load_super(midx, par):
            sb = wid * NSUPER + midx
            pltpu.sync_copy(rc_hbm.at[sb], rcs[par])
            pltpu.sync_copy(vals_hbm.at[sb], vls[par])

        def g_idx(par, u):
            return rcs[par].at[2 * u + 1]

        def w_idx(par, u):
            return rcs[par].at[2 * u]

        def start_gather(par, u, b):
            pltpu.async_copy(h_hbm.at[g_idx(par, u)], gats[b], sgs[b])

        def wait_gather(par, u, b):
            pltpu.make_async_copy(h_hbm.at[g_idx(par, u)], gats[b],
                                  sgs[b]).wait()

        def start_scatter(par, u, b):
            pltpu.async_copy(gats[b], acc_sh.at[w_idx(par, u)], sss[b],
                             add=True)

        def wait_scatter(par, u, b):
            pltpu.make_async_copy(gats[b], acc_sh.at[w_idx(par, u)],
                                  sss[b]).wait()

        def scale(par, u, b):
            gat = gats[b]
            vv = vls[par]

            def ebody(e, carry):
                v16 = jnp.broadcast_to(vv[u, pl.ds(e, 16)][0], (16,))
                for q in range(D // 16):
                    sl = pl.ds(q * 16, 16)
                    gat[e, sl] = gat[e, sl] * v16
                return carry

            lax.fori_loop(0, CH, ebody, 0)

        last_scat = {}

        def emit_chunk(j, i_var=None, with_next=True, with_super=True):
            par, u, b = _coords(j)
            wait_gather(par, u, b)
            if with_next:
                parn, un, bn = _coords(j + 1)
                if bn in last_scat:
                    wait_scatter(*last_scat[bn], bn)
                if with_super and u == 2:
                    m1 = j // SB + 1
                    midx = m1 if i_var is None else 6 * i_var + m1
                    load_super(midx, m1 % 2)
                start_gather(parn, un, bn)
            scale(par, u, b)
            start_scatter(par, u, b)
            last_scat[b] = (par, u)

        load_super(0, 0)
        start_gather(0, 0, 0)
        emit_chunk(0, with_super=False)
        emit_chunk(1, with_super=False)

        def pipe_body(i, carry):
            for off in range(PERIOD):
                emit_chunk(STEADY_START + off, i_var=i)
            return carry

        lax.fori_loop(0, STEADY_ITERS, pipe_body, 0)

        emit_chunk(122, with_super=False)
        emit_chunk(123, with_super=False)
        emit_chunk(124, with_next=False)

        for b in range(3):
            wait_scatter(*last_scat[b], b)

        plsc.subcore_barrier()

        @pl.when(c == 0)
        def _():
            pltpu.sync_copy(acc_sh.at[pl.ds(s * RPS, RPS)],
                            part0_hbm.at[pl.ds(s * RPS, RPS)])

        @pl.when(c == 1)
        def _():
            pltpu.sync_copy(acc_sh.at[pl.ds(s * RPS, RPS)],
                            part1_hbm.at[pl.ds(s * RPS, RPS)])

        @pl.when((s == NS - 1) & (c == 0))
        def _():
            pltpu.sync_copy(acc_sh.at[pl.ds(NS * RPS, RREM)],
                            part0_hbm.at[pl.ds(NS * RPS, RREM)])

        @pl.when((s == NS - 1) & (c == 1))
        def _():
            pltpu.sync_copy(acc_sh.at[pl.ds(NS * RPS, RREM)],
                            part1_hbm.at[pl.ds(NS * RPS, RREM)])

    return body(h, rc2, vals2, perm, re, zblk)


def _tc_final(p0, p1, left, w, b):
    def body(p0_ref, p1_ref, l_ref, w_ref, b_ref, o_ref):
        lft = l_ref[...]
        g = jax.nn.sigmoid(
            jnp.dot(lft, w_ref[...], preferred_element_type=jnp.float32)
            + b_ref[...])
        p = jnp.maximum(p0_ref[...] + p1_ref[...], 0.0)
        o_ref[...] = g * p + (1.0 - g) * lft

    return pl.pallas_call(
        body,
        grid=(N // 1000,),
        in_specs=[pl.BlockSpec((1000, D), lambda i: (i, 0)),
                  pl.BlockSpec((1000, D), lambda i: (i, 0)),
                  pl.BlockSpec((1000, D), lambda i: (i, 0)),
                  pl.BlockSpec((D, D), lambda i: (0, 0)),
                  pl.BlockSpec((1, D), lambda i: (0, 0))],
        out_specs=pl.BlockSpec((1000, D), lambda i: (i, 0)),
        out_shape=jax.ShapeDtypeStruct((N, D), jnp.float32),
    )(p0, p1, left, w, b)


def kernel(right_embed, edge_index, adj_vals, perm, gcnW1,
           highwayWr, highwaybr):
    ei = edge_index.astype(jnp.int32)
    r4 = ei[0].reshape(NW * NSUPER, SB, 1, CH)
    c4 = ei[1].reshape(NW * NSUPER, SB, 1, CH)
    rc2 = jnp.concatenate([r4, c4], axis=2).reshape(NW * NSUPER,
                                                    2 * SB, CH)
    vals2 = jnp.pad(adj_vals.reshape(NW * NSUPER, SB, CH),
                    ((0, 0), (0, 0), (0, 16)))
    zblk = jnp.zeros((RPS, D), jnp.float32)

    h = _tc_matmul(right_embed, gcnW1)
    part0, part1, left = _sc_spmm(h, rc2, vals2,
                                  perm.astype(jnp.int32), right_embed, zblk)
    return _tc_final(part0, part1, left, highwayWr,
                     highwaybr.reshape(1, D))
